# baseline (speedup 1.0000x reference)
import functools

import jax
import jax.numpy as jnp
import numpy as np
from jax import lax
from jax.experimental import pallas as pl
from jax.experimental.pallas import tpu as pltpu
from jax.experimental.pallas import tpu_sc as plsc

VOCAB = 1000000
D_EMBED = 64
L_SEQ = 200
BATCH = 1024
SCALE = 8.0

NUM_CORES = 2
NUM_SUBCORES = 16
NUM_WORKERS = NUM_CORES * NUM_SUBCORES
ROWS_PER_WORKER = BATCH * L_SEQ // NUM_WORKERS
CHUNK = 256
N_CHUNK = ROWS_PER_WORKER // CHUNK
SEQ_PER_EPI_BLOCK = 4


def _pe_const() -> jnp.ndarray:
    pos = np.arange(L_SEQ, dtype=np.float32)[:, None]
    wavelengths = np.exp(
        np.arange(0, D_EMBED, 2, dtype=np.float32) / D_EMBED * -np.log(10000.0)
    )
    pe = np.zeros((L_SEQ, D_EMBED), dtype=np.float32)
    pe[:, 0::2] = np.sin(pos * wavelengths)
    pe[:, 1::2] = np.cos(pos * wavelengths)
    return jnp.asarray(pe)


_MESH = plsc.VectorSubcoreMesh(core_axis_name="c", subcore_axis_name="s")


@functools.partial(
    pl.kernel,
    mesh=_MESH,
    out_type=jax.ShapeDtypeStruct((BATCH * L_SEQ, D_EMBED), jnp.float32),
    scratch_types=[
        pltpu.VMEM((CHUNK,), jnp.int32),
        pltpu.VMEM((CHUNK,), jnp.int32),
        pltpu.VMEM((CHUNK, D_EMBED), jnp.float32),
        pltpu.VMEM((CHUNK, D_EMBED), jnp.float32),
        pltpu.SemaphoreType.DMA,
        pltpu.SemaphoreType.DMA,
        pltpu.SemaphoreType.DMA,
        pltpu.SemaphoreType.DMA,
        pltpu.SemaphoreType.DMA,
        pltpu.SemaphoreType.DMA,
    ],
)
def _gather_sc(x_hbm, t3_hbm, out_hbm, i0, i1, b0, b1, si0, si1, sg0, sg1, sw0, sw1):
    wid = lax.axis_index("s") * NUM_CORES + lax.axis_index("c")
    base = wid * ROWS_PER_WORKER
    idx, bufs = (i0, i1), (b0, b1)
    SI, SG, SW = (si0, si1), (sg0, sg1), (sw0, sw1)

    def start_idx(c, b):
        pltpu.async_copy(x_hbm.at[pl.ds(base + c * CHUNK, CHUNK)], idx[b], SI[b])

    def wait_idx(b):
        pltpu.make_async_copy(x_hbm.at[pl.ds(0, CHUNK)], idx[b], SI[b]).wait()

    def issue_gather(b):
        def slab(s, carry):
            vv = idx[b][pl.ds(s * 16, 16)]
            for j in range(16):
                v = vv[j]
                pltpu.async_copy(
                    t3_hbm.at[
                        lax.shift_right_logical(v, 3),
                        pl.ds(lax.bitwise_and(v, 7), 1),
                        :,
                    ],
                    bufs[b].at[pl.ds(s * 16 + j, 1)],
                    SG[b],
                )
            return carry

        lax.fori_loop(0, CHUNK // 16, slab, 0)

    def wait_gather(b):
        pltpu.make_async_copy(
            out_hbm.at[pl.ds(0, CHUNK)], bufs[b], SG[b]
        ).wait()

    def start_out(c, b):
        pltpu.async_copy(
            bufs[b], out_hbm.at[pl.ds(base + c * CHUNK, CHUNK)], SW[b]
        )

    def wait_out(b):
        pltpu.make_async_copy(bufs[0], out_hbm.at[pl.ds(0, CHUNK)], SW[b]).wait()

    start_idx(0, 0)
    start_idx(1, 1)
    wait_idx(0)
    issue_gather(0)
    start_idx(2, 0)

    for c in range(N_CHUNK):
        b = c % 2
        b1 = (c + 1) % 2
        if c + 1 < N_CHUNK:
            wait_idx(b1)
            if c >= 1:
                wait_out(b1)
            issue_gather(b1)
            if c + 3 < N_CHUNK:
                start_idx(c + 3, b1)
        wait_gather(b)
        start_out(c, b)
    wait_out((N_CHUNK - 1) % 2)


def _epi_body(raw_ref, pe_ref, out_ref):
    out_ref[...] = raw_ref[...] * SCALE + pe_ref[...]


_EPI_ROWS = L_SEQ * SEQ_PER_EPI_BLOCK


_epi_tc = pl.pallas_call(
    _epi_body,
    grid=(BATCH // SEQ_PER_EPI_BLOCK,),
    in_specs=[
        pl.BlockSpec((_EPI_ROWS, D_EMBED), lambda i: (i, 0)),
        pl.BlockSpec((_EPI_ROWS, D_EMBED), lambda i: (0, 0)),
    ],
    out_specs=pl.BlockSpec((_EPI_ROWS, D_EMBED), lambda i: (i, 0)),
    out_shape=jax.ShapeDtypeStruct((BATCH * L_SEQ, D_EMBED), jnp.float32),
)


@jax.jit
def kernel(x, table):
    xf = x.reshape(-1).astype(jnp.int32)
    t3 = table.reshape(VOCAB // 8, 8, D_EMBED)
    raw = _gather_sc(xf, t3)
    pe_tiled = jnp.tile(_pe_const(), (SEQ_PER_EPI_BLOCK, 1))
    out = _epi_tc(raw, pe_tiled)
    return out.reshape(BATCH, L_SEQ, D_EMBED)


# speedup vs baseline: 1.2375x; 1.2375x over previous
import functools

import jax
import jax.numpy as jnp
import numpy as np
from jax import lax
from jax.experimental import pallas as pl
from jax.experimental.pallas import tpu as pltpu
from jax.experimental.pallas import tpu_sc as plsc

VOCAB = 1000000
D_EMBED = 64
L_SEQ = 200
BATCH = 1024
SCALE = 8.0

NUM_CORES = 2
NUM_SUBCORES = 16
NUM_WORKERS = NUM_CORES * NUM_SUBCORES
ROWS_PER_WORKER = BATCH * L_SEQ // NUM_WORKERS
CHUNK = 256
N_CHUNK = ROWS_PER_WORKER // CHUNK
SEQ_PER_EPI_BLOCK = 16


def _pe_const() -> jnp.ndarray:
    pos = np.arange(L_SEQ, dtype=np.float32)[:, None]
    wavelengths = np.exp(
        np.arange(0, D_EMBED, 2, dtype=np.float32) / D_EMBED * -np.log(10000.0)
    )
    pe = np.zeros((L_SEQ, D_EMBED), dtype=np.float32)
    pe[:, 0::2] = np.sin(pos * wavelengths)
    pe[:, 1::2] = np.cos(pos * wavelengths)
    return jnp.asarray(pe)


_MESH = plsc.VectorSubcoreMesh(core_axis_name="c", subcore_axis_name="s")


@functools.partial(
    pl.kernel,
    mesh=_MESH,
    out_type=jax.ShapeDtypeStruct((BATCH * L_SEQ, D_EMBED), jnp.float32),
    scratch_types=[
        pltpu.VMEM((CHUNK,), jnp.int32),
        pltpu.VMEM((CHUNK,), jnp.int32),
        pltpu.VMEM((CHUNK, D_EMBED), jnp.float32),
        pltpu.VMEM((CHUNK, D_EMBED), jnp.float32),
        pltpu.SemaphoreType.DMA,
        pltpu.SemaphoreType.DMA,
        pltpu.SemaphoreType.DMA,
        pltpu.SemaphoreType.DMA,
        pltpu.SemaphoreType.DMA,
        pltpu.SemaphoreType.DMA,
    ],
)
def _gather_sc(x_hbm, t3_hbm, out_hbm, i0, i1, b0, b1, si0, si1, sg0, sg1, sw0, sw1):
    wid = lax.axis_index("s") * NUM_CORES + lax.axis_index("c")
    base = wid * ROWS_PER_WORKER
    idx, bufs = (i0, i1), (b0, b1)
    SI, SG, SW = (si0, si1), (sg0, sg1), (sw0, sw1)

    def start_idx(c, b):
        pltpu.async_copy(x_hbm.at[pl.ds(base + c * CHUNK, CHUNK)], idx[b], SI[b])

    def wait_idx(b):
        pltpu.make_async_copy(x_hbm.at[pl.ds(0, CHUNK)], idx[b], SI[b]).wait()

    def issue_gather(b):
        def slab(s, carry):
            vv = idx[b][pl.ds(s * 16, 16)]
            for j in range(16):
                v = vv[j]
                pltpu.async_copy(
                    t3_hbm.at[
                        lax.shift_right_logical(v, 3),
                        pl.ds(lax.bitwise_and(v, 7), 1),
                        :,
                    ],
                    bufs[b].at[pl.ds(s * 16 + j, 1)],
                    SG[b],
                )
            return carry

        lax.fori_loop(0, CHUNK // 16, slab, 0)

    def wait_gather(b):
        pltpu.make_async_copy(
            out_hbm.at[pl.ds(0, CHUNK)], bufs[b], SG[b]
        ).wait()

    def start_out(c, b):
        pltpu.async_copy(
            bufs[b], out_hbm.at[pl.ds(base + c * CHUNK, CHUNK)], SW[b]
        )

    def wait_out(b):
        pltpu.make_async_copy(bufs[0], out_hbm.at[pl.ds(0, CHUNK)], SW[b]).wait()

    start_idx(0, 0)
    start_idx(1, 1)
    wait_idx(0)
    issue_gather(0)
    start_idx(2, 0)

    for c in range(N_CHUNK):
        b = c % 2
        b1 = (c + 1) % 2
        if c + 1 < N_CHUNK:
            wait_idx(b1)
            if c >= 1:
                wait_out(b1)
            issue_gather(b1)
            if c + 3 < N_CHUNK:
                start_idx(c + 3, b1)
        wait_gather(b)
        start_out(c, b)
    wait_out((N_CHUNK - 1) % 2)


def _epi_body(raw_ref, pe_ref, out_ref):
    out_ref[...] = raw_ref[...] * SCALE + pe_ref[...]


_EPI_ROWS = L_SEQ * SEQ_PER_EPI_BLOCK


_epi_tc = pl.pallas_call(
    _epi_body,
    grid=(BATCH // SEQ_PER_EPI_BLOCK,),
    in_specs=[
        pl.BlockSpec((_EPI_ROWS, D_EMBED), lambda i: (i, 0)),
        pl.BlockSpec((_EPI_ROWS, D_EMBED), lambda i: (0, 0)),
    ],
    out_specs=pl.BlockSpec((_EPI_ROWS, D_EMBED), lambda i: (i, 0)),
    out_shape=jax.ShapeDtypeStruct((BATCH * L_SEQ, D_EMBED), jnp.float32),
)


@jax.jit
def kernel(x, table):
    xf = x.reshape(-1).astype(jnp.int32)
    t3 = table.reshape(VOCAB // 8, 8, D_EMBED)
    raw = _gather_sc(xf, t3)
    pe_tiled = jnp.tile(_pe_const(), (SEQ_PER_EPI_BLOCK, 1))
    out = _epi_tc(raw, pe_tiled)
    return out.reshape(BATCH, L_SEQ, D_EMBED)


# speedup vs baseline: 1.2389x; 1.0011x over previous
import functools

import jax
import jax.numpy as jnp
import numpy as np
from jax import lax
from jax.experimental import pallas as pl
from jax.experimental.pallas import tpu as pltpu
from jax.experimental.pallas import tpu_sc as plsc

VOCAB = 1000000
D_EMBED = 64
L_SEQ = 200
BATCH = 1024
SCALE = 8.0

NUM_CORES = 2
NUM_SUBCORES = 16
NUM_WORKERS = NUM_CORES * NUM_SUBCORES
HALF_ROWS = BATCH * L_SEQ // 2
ROWS_PER_WORKER = HALF_ROWS // NUM_WORKERS
CHUNK = 320
N_CHUNK = ROWS_PER_WORKER // CHUNK
SEQ_PER_EPI_BLOCK = 16
_EPI_ROWS = L_SEQ * SEQ_PER_EPI_BLOCK
_EPI_BLOCKS_PER_HALF = HALF_ROWS // _EPI_ROWS


def _pe_const() -> jnp.ndarray:
    pos = np.arange(L_SEQ, dtype=np.float32)[:, None]
    wavelengths = np.exp(
        np.arange(0, D_EMBED, 2, dtype=np.float32) / D_EMBED * -np.log(10000.0)
    )
    pe = np.zeros((L_SEQ, D_EMBED), dtype=np.float32)
    pe[:, 0::2] = np.sin(pos * wavelengths)
    pe[:, 1::2] = np.cos(pos * wavelengths)
    return jnp.asarray(pe)


_MESH = plsc.VectorSubcoreMesh(core_axis_name="c", subcore_axis_name="s")


@functools.partial(
    pl.kernel,
    mesh=_MESH,
    out_type=jax.ShapeDtypeStruct((HALF_ROWS, D_EMBED), jnp.float32),
    scratch_types=[
        pltpu.VMEM((CHUNK,), jnp.int32),
        pltpu.VMEM((CHUNK,), jnp.int32),
        pltpu.VMEM((CHUNK, D_EMBED), jnp.float32),
        pltpu.VMEM((CHUNK, D_EMBED), jnp.float32),
        pltpu.SemaphoreType.DMA,
        pltpu.SemaphoreType.DMA,
        pltpu.SemaphoreType.DMA,
        pltpu.SemaphoreType.DMA,
        pltpu.SemaphoreType.DMA,
        pltpu.SemaphoreType.DMA,
    ],
)
def _gather_sc(x_hbm, t3_hbm, out_hbm, i0, i1, b0, b1, si0, si1, sg0, sg1, sw0, sw1):
    wid = lax.axis_index("s") * NUM_CORES + lax.axis_index("c")
    base = wid * ROWS_PER_WORKER
    idx, bufs = (i0, i1), (b0, b1)
    SI, SG, SW = (si0, si1), (sg0, sg1), (sw0, sw1)

    def start_idx(c, b):
        pltpu.async_copy(x_hbm.at[pl.ds(base + c * CHUNK, CHUNK)], idx[b], SI[b])

    def wait_idx(b):
        pltpu.make_async_copy(x_hbm.at[pl.ds(0, CHUNK)], idx[b], SI[b]).wait()

    def issue_gather(b):
        def slab(s, carry):
            vv = idx[b][pl.ds(s * 16, 16)]
            for j in range(16):
                v = vv[j]
                pltpu.async_copy(
                    t3_hbm.at[
                        lax.shift_right_logical(v, 3),
                        pl.ds(lax.bitwise_and(v, 7), 1),
                        :,
                    ],
                    bufs[b].at[pl.ds(s * 16 + j, 1)],
                    SG[b],
                )
            return carry

        lax.fori_loop(0, CHUNK // 16, slab, 0)

    def wait_gather(b):
        pltpu.make_async_copy(out_hbm.at[pl.ds(0, CHUNK)], bufs[b], SG[b]).wait()

    def start_out(c, b):
        pltpu.async_copy(bufs[b], out_hbm.at[pl.ds(base + c * CHUNK, CHUNK)], SW[b])

    def wait_out(b):
        pltpu.make_async_copy(bufs[0], out_hbm.at[pl.ds(0, CHUNK)], SW[b]).wait()

    start_idx(0, 0)
    start_idx(1, 1)
    wait_idx(0)
    issue_gather(0)
    start_idx(2, 0)

    for c in range(N_CHUNK):
        b = c % 2
        b1 = (c + 1) % 2
        if c + 1 < N_CHUNK:
            wait_idx(b1)
            if c >= 1:
                wait_out(b1)
            issue_gather(b1)
            if c + 3 < N_CHUNK:
                start_idx(c + 3, b1)
        wait_gather(b)
        start_out(c, b)
    wait_out((N_CHUNK - 1) % 2)


def _epi_lo_body(raw_ref, pe_ref, out_ref):
    out_ref[...] = raw_ref[...] * SCALE + pe_ref[...]


_epi_lo = pl.pallas_call(
    _epi_lo_body,
    grid=(_EPI_BLOCKS_PER_HALF,),
    in_specs=[
        pl.BlockSpec((_EPI_ROWS, D_EMBED), lambda i: (i, 0)),
        pl.BlockSpec((_EPI_ROWS, D_EMBED), lambda i: (0, 0)),
    ],
    out_specs=pl.BlockSpec((_EPI_ROWS, D_EMBED), lambda i: (i, 0)),
    out_shape=jax.ShapeDtypeStruct((BATCH * L_SEQ, D_EMBED), jnp.float32),
)


def _epi_hi_body(raw_ref, pe_ref, acc_ref, out_ref):
    del acc_ref
    out_ref[...] = raw_ref[...] * SCALE + pe_ref[...]


_epi_hi = pl.pallas_call(
    _epi_hi_body,
    grid=(_EPI_BLOCKS_PER_HALF,),
    in_specs=[
        pl.BlockSpec((_EPI_ROWS, D_EMBED), lambda i: (i, 0)),
        pl.BlockSpec((_EPI_ROWS, D_EMBED), lambda i: (0, 0)),
        pl.BlockSpec(memory_space=pl.ANY),
    ],
    out_specs=pl.BlockSpec(
        (_EPI_ROWS, D_EMBED), lambda i: (i + _EPI_BLOCKS_PER_HALF, 0)
    ),
    out_shape=jax.ShapeDtypeStruct((BATCH * L_SEQ, D_EMBED), jnp.float32),
    input_output_aliases={2: 0},
)


@jax.jit
def kernel(x, table):
    xf = x.reshape(-1).astype(jnp.int32)
    t3 = table.reshape(VOCAB // 8, 8, D_EMBED)
    raw_lo = _gather_sc(xf[:HALF_ROWS], t3)
    raw_hi = _gather_sc(xf[HALF_ROWS:], t3)
    pe_tiled = jnp.tile(_pe_const(), (SEQ_PER_EPI_BLOCK, 1))
    o1 = _epi_lo(raw_lo, pe_tiled)
    out = _epi_hi(raw_hi, pe_tiled, o1)
    return out.reshape(BATCH, L_SEQ, D_EMBED)


# speedup vs baseline: 1.2486x; 1.0078x over previous
import functools

import jax
import jax.numpy as jnp
import numpy as np
from jax import lax
from jax.experimental import pallas as pl
from jax.experimental.pallas import tpu as pltpu
from jax.experimental.pallas import tpu_sc as plsc

VOCAB = 1000000
D_EMBED = 64
L_SEQ = 200
BATCH = 1024
SCALE = 8.0

NUM_CORES = 2
NUM_SUBCORES = 16
NUM_WORKERS = NUM_CORES * NUM_SUBCORES
ROWS_PER_WORKER = BATCH * L_SEQ // NUM_WORKERS
CHUNK = 400
N_CHUNK = ROWS_PER_WORKER // CHUNK
SEQ_PER_EPI_BLOCK = 32


def _pe_const() -> jnp.ndarray:
    pos = np.arange(L_SEQ, dtype=np.float32)[:, None]
    wavelengths = np.exp(
        np.arange(0, D_EMBED, 2, dtype=np.float32) / D_EMBED * -np.log(10000.0)
    )
    pe = np.zeros((L_SEQ, D_EMBED), dtype=np.float32)
    pe[:, 0::2] = np.sin(pos * wavelengths)
    pe[:, 1::2] = np.cos(pos * wavelengths)
    return jnp.asarray(pe)


_MESH = plsc.VectorSubcoreMesh(core_axis_name="c", subcore_axis_name="s")


@functools.partial(
    pl.kernel,
    mesh=_MESH,
    out_type=jax.ShapeDtypeStruct((BATCH * L_SEQ, D_EMBED), jnp.float32),
    scratch_types=[
        pltpu.VMEM((CHUNK,), jnp.int32),
        pltpu.VMEM((CHUNK,), jnp.int32),
        pltpu.VMEM((CHUNK, D_EMBED), jnp.float32),
        pltpu.VMEM((CHUNK, D_EMBED), jnp.float32),
        pltpu.SemaphoreType.DMA,
        pltpu.SemaphoreType.DMA,
        pltpu.SemaphoreType.DMA,
        pltpu.SemaphoreType.DMA,
        pltpu.SemaphoreType.DMA,
        pltpu.SemaphoreType.DMA,
    ],
)
def _gather_sc(x_hbm, t3_hbm, out_hbm, i0, i1, b0, b1, si0, si1, sg0, sg1, sw0, sw1):
    wid = lax.axis_index("s") * NUM_CORES + lax.axis_index("c")
    base = wid * ROWS_PER_WORKER
    idx, bufs = (i0, i1), (b0, b1)
    SI, SG, SW = (si0, si1), (sg0, sg1), (sw0, sw1)

    def start_idx(c, b):
        pltpu.async_copy(x_hbm.at[pl.ds(base + c * CHUNK, CHUNK)], idx[b], SI[b])

    def wait_idx(b):
        pltpu.make_async_copy(x_hbm.at[pl.ds(0, CHUNK)], idx[b], SI[b]).wait()

    def issue_gather(b):
        def slab(s, carry):
            vv = idx[b][pl.ds(s * 16, 16)]
            for j in range(16):
                v = vv[j]
                pltpu.async_copy(
                    t3_hbm.at[
                        lax.shift_right_logical(v, 3),
                        pl.ds(lax.bitwise_and(v, 7), 1),
                        :,
                    ],
                    bufs[b].at[pl.ds(s * 16 + j, 1)],
                    SG[b],
                )
            return carry

        lax.fori_loop(0, CHUNK // 16, slab, 0)

    def wait_gather(b):
        pltpu.make_async_copy(
            out_hbm.at[pl.ds(0, CHUNK)], bufs[b], SG[b]
        ).wait()

    def start_out(c, b):
        pltpu.async_copy(
            bufs[b], out_hbm.at[pl.ds(base + c * CHUNK, CHUNK)], SW[b]
        )

    def wait_out(b):
        pltpu.make_async_copy(bufs[0], out_hbm.at[pl.ds(0, CHUNK)], SW[b]).wait()

    start_idx(0, 0)
    start_idx(1, 1)
    wait_idx(0)
    issue_gather(0)
    start_idx(2, 0)

    for c in range(N_CHUNK):
        b = c % 2
        b1 = (c + 1) % 2
        if c + 1 < N_CHUNK:
            wait_idx(b1)
            if c >= 1:
                wait_out(b1)
            issue_gather(b1)
            if c + 3 < N_CHUNK:
                start_idx(c + 3, b1)
        wait_gather(b)
        start_out(c, b)
    wait_out((N_CHUNK - 1) % 2)


def _epi_body(raw_ref, pe_ref, out_ref):
    out_ref[...] = raw_ref[...] * SCALE + pe_ref[...]


_EPI_ROWS = L_SEQ * SEQ_PER_EPI_BLOCK


_epi_tc = pl.pallas_call(
    _epi_body,
    grid=(BATCH // SEQ_PER_EPI_BLOCK,),
    in_specs=[
        pl.BlockSpec((_EPI_ROWS, D_EMBED), lambda i: (i, 0)),
        pl.BlockSpec((_EPI_ROWS, D_EMBED), lambda i: (0, 0)),
    ],
    out_specs=pl.BlockSpec((_EPI_ROWS, D_EMBED), lambda i: (i, 0)),
    out_shape=jax.ShapeDtypeStruct((BATCH * L_SEQ, D_EMBED), jnp.float32),
)


@jax.jit
def kernel(x, table):
    xf = x.reshape(-1).astype(jnp.int32)
    t3 = table.reshape(VOCAB // 8, 8, D_EMBED)
    raw = _gather_sc(xf, t3)
    pe_tiled = jnp.tile(_pe_const(), (SEQ_PER_EPI_BLOCK, 1))
    out = _epi_tc(raw, pe_tiled)
    return out.reshape(BATCH, L_SEQ, D_EMBED)
